# baseline (speedup 1.0000x reference)
import functools

import jax
import jax.numpy as jnp
import numpy as np
from jax import lax
from jax.experimental import pallas as pl
from jax.experimental.pallas import tpu as pltpu
from jax.experimental.pallas import tpu_sc as plsc

EPS = 1e-5
BS = 4096
D = 512
K = 3
P3 = BS * K

_W_NP = np.random.default_rng(0).random(P3).astype(np.float32)
_W_SPLAT_NP = np.repeat(_W_NP[:, None], 16, axis=1)

def _dot_t(a, b):
    return lax.dot_general(a.astype(jnp.bfloat16), b.astype(jnp.bfloat16),
                           (((1,), (1,)), ((), ())),
                           preferred_element_type=jnp.float32)


def _bn_swish(x, g, b):
    m = jnp.mean(x, axis=0, keepdims=True)
    v = jnp.mean((x - m) ** 2, axis=0, keepdims=True)
    y = (x - m) / jnp.sqrt(v + EPS) * g + b
    return y * jax.nn.sigmoid(y)


def _transform_body(fea_ref, w1_ref, b1_ref, g1_ref, be1_ref,
                    w2_ref, b2_ref, g2_ref, be2_ref, h_ref, xxr_ref):
    h1 = _dot_t(fea_ref[...], w1_ref[...]) + b1_ref[...]
    h1 = _bn_swish(h1, g1_ref[...], be1_ref[...])
    h2 = _dot_t(h1, w2_ref[...]) + b2_ref[...]
    h2 = _bn_swish(h2, g2_ref[...], be2_ref[...])
    h_ref[...] = h2
    xxr_ref[...] = jnp.sum(h2 * h2, axis=1).reshape(1, BS)


def _knn_body(h_blk_ref, h_all_ref, xxr_ref, idx_ref):
    hb = h_blk_ref[...]
    gram = _dot_t(hb, h_all_ref[...])
    inner = -2.0 * gram
    xxb = jnp.sum(hb * hb, axis=1, keepdims=True)
    s = (-xxb - inner) - xxr_ref[...]
    iota = lax.broadcasted_iota(jnp.int32, s.shape, 1)
    lane = lax.broadcasted_iota(jnp.int32, (s.shape[0], 128), 1)
    picks = []
    for k in range(K):
        m = jnp.max(s, axis=1, keepdims=True)
        cand = jnp.where(s == m, iota, BS)
        ik = jnp.min(cand, axis=1, keepdims=True)
        picks.append(ik)
        if k < K - 1:
            s = jnp.where(iota == ik, -jnp.inf, s)
    out = jnp.where(lane == 0, picks[0],
                    jnp.where(lane == 1, picks[1],
                              jnp.where(lane == 2, picks[2], 0)))
    idx_ref[...] = out


_NW = 32
_CHUNK = 48
_ROWS_PER_W = P3 // _NW
_NCHUNK = _ROWS_PER_W // _CHUNK
_HROWS_PER_W = BS // _NW
_NV = D // 16


def _smote_body(h_hbm, idxt_hbm, w_hbm, out_hbm,
                bidx_all, bidx_flat, w_all, b0, b1, o0, o1, sem, semo, semh):
    wid = lax.axis_index("s") * 2 + lax.axis_index("c")
    hbase = wid * _HROWS_PER_W
    sbase = wid * _ROWS_PER_W

    cph = pltpu.async_copy(h_hbm.at[pl.ds(hbase, _HROWS_PER_W)],
                           out_hbm.at[pl.ds(hbase, _HROWS_PER_W)], semh)

    for t in range(3):
        pltpu.sync_copy(idxt_hbm.at[pl.ds(t * BS + hbase, _HROWS_PER_W)],
                        bidx_all.at[pl.ds(t * _HROWS_PER_W, _HROWS_PER_W)])
    pltpu.sync_copy(w_hbm.at[pl.ds(sbase * 16, _ROWS_PER_W * 16)], w_all)

    for c in range(_NCHUNK):
        for t in range(3):
            bidx_flat[pl.ds(c * _CHUNK + 16 * t, 16)] = \
                bidx_all[pl.ds(t * _HROWS_PER_W + c * 16, 16)]

    def issue(c, buf):
        return pltpu.async_copy(h_hbm.at[bidx_flat.at[pl.ds(c * _CHUNK,
                                                            _CHUNK)]],
                                buf, sem)

    def compute(c, buf, obuf):
        def anchor(aj, _):
            aregs = [buf[aj, pl.ds(16 * v, 16)] for v in range(_NV)]
            for v in range(_NV):
                obuf[aj * 3, pl.ds(16 * v, 16)] = aregs[v]
            for t in range(1, 3):
                j = aj * 3 + t
                wrow = w_all[pl.ds((c * _CHUNK + j) * 16, 16)]
                for v in range(_NV):
                    sl = pl.ds(16 * v, 16)
                    b = buf[16 * t + aj, sl]
                    obuf[j, sl] = aregs[v] + wrow * (b - aregs[v])
            return 0

        lax.fori_loop(0, 16, anchor, 0)
        return pltpu.async_copy(obuf,
                                out_hbm.at[pl.ds(BS + sbase + c * _CHUNK,
                                                 _CHUNK)], semo)

    def drain_gather(buf):
        pltpu.make_async_copy(h_hbm.at[pl.ds(0, _CHUNK)], buf, sem).wait()

    def drain_out(obuf):
        pltpu.make_async_copy(obuf, out_hbm.at[pl.ds(BS, _CHUNK)],
                              semo).wait()

    issue(0, b0)

    def pair(p, _):
        c0 = 2 * p
        issue(c0 + 1, b1)
        drain_gather(b0)

        @pl.when(p > 0)
        def _():
            drain_out(o0)

        compute(c0, b0, o0)

        @pl.when(p < _NCHUNK // 2 - 1)
        def _():
            issue(c0 + 2, b0)

        drain_gather(b1)

        @pl.when(p > 0)
        def _():
            drain_out(o1)

        compute(c0 + 1, b1, o1)
        return 0

    lax.fori_loop(0, _NCHUNK // 2, pair, 0)
    drain_out(o0)
    drain_out(o1)
    cph.wait()


@jax.jit
def _pipeline(fea, lbl, W1, b1, g1, be1, W2, b2, g2, be2):
    vecs = [v.reshape(1, D) for v in (b1, g1, be1, b2, g2, be2)]
    b1r, g1r, be1r, b2r, g2r, be2r = vecs

    h, xxr = pl.pallas_call(
        _transform_body,
        out_shape=(jax.ShapeDtypeStruct((BS, D), jnp.float32),
                   jax.ShapeDtypeStruct((1, BS), jnp.float32)),
    )(fea, W1, b1r, g1r, be1r, W2, b2r, g2r, be2r)

    nblk = 16
    blk = BS // nblk
    idx_pad = pl.pallas_call(
        _knn_body,
        grid=(nblk,),
        in_specs=[
            pl.BlockSpec((blk, D), lambda i: (i, 0)),
            pl.BlockSpec((BS, D), lambda i: (0, 0)),
            pl.BlockSpec((1, BS), lambda i: (0, 0)),
        ],
        out_specs=pl.BlockSpec((blk, 128), lambda i: (i, 0)),
        out_shape=jax.ShapeDtypeStruct((BS, 128), jnp.int32),
    )(h, h, xxr)

    idxt = jnp.transpose(idx_pad[:, :K])

    mesh = plsc.VectorSubcoreMesh(core_axis_name="c", subcore_axis_name="s")
    smote = pl.kernel(
        _smote_body,
        mesh=mesh,
        out_type=jax.ShapeDtypeStruct((BS + P3, D), jnp.float32),
        scratch_types=[
            pltpu.VMEM((3 * _HROWS_PER_W,), jnp.int32),
            pltpu.VMEM((_ROWS_PER_W,), jnp.int32),
            pltpu.VMEM((_ROWS_PER_W * 16,), jnp.float32),
            pltpu.VMEM((_CHUNK, D), jnp.float32),
            pltpu.VMEM((_CHUNK, D), jnp.float32),
            pltpu.VMEM((_CHUNK, D), jnp.float32),
            pltpu.VMEM((_CHUNK, D), jnp.float32),
            pltpu.SemaphoreType.DMA,
            pltpu.SemaphoreType.DMA,
            pltpu.SemaphoreType.DMA,
        ],
    )
    fea_out = smote(h, idxt.reshape(-1), jnp.asarray(_W_SPLAT_NP.reshape(-1)))

    lbl_out = jnp.concatenate([lbl, jnp.ones((P3, 1), jnp.float32)], axis=0)
    return fea_out, lbl_out


def kernel(fea, lbl, W1, b1, g1, be1, W2, b2, g2, be2):
    return _pipeline(fea, lbl, W1, b1, g1, be1, W2, b2, g2, be2)


# speedup vs baseline: 1.9197x; 1.9197x over previous
import functools

import jax
import jax.numpy as jnp
import numpy as np
from jax import lax
from jax.experimental import pallas as pl
from jax.experimental.pallas import tpu as pltpu
from jax.experimental.pallas import tpu_sc as plsc

EPS = 1e-5
BS = 4096
D = 512
K = 3
P3 = BS * K

_W_NP = np.random.default_rng(0).random(P3).astype(np.float32)
_W_SPLAT_NP = np.repeat(_W_NP[:, None], 16, axis=1)

def _dot_t(a, b):
    return lax.dot_general(a.astype(jnp.bfloat16), b.astype(jnp.bfloat16),
                           (((1,), (1,)), ((), ())),
                           preferred_element_type=jnp.float32)


def _bn_swish(x, g, b):
    m = jnp.mean(x, axis=0, keepdims=True)
    v = jnp.mean((x - m) ** 2, axis=0, keepdims=True)
    y = (x - m) / jnp.sqrt(v + EPS) * g + b
    return y * jax.nn.sigmoid(y)


def _transform_body(fea_ref, w1_ref, b1_ref, g1_ref, be1_ref,
                    w2_ref, b2_ref, g2_ref, be2_ref, h_ref, xxr_ref):
    h1 = _dot_t(fea_ref[...], w1_ref[...]) + b1_ref[...]
    h1 = _bn_swish(h1, g1_ref[...], be1_ref[...])
    h2 = _dot_t(h1, w2_ref[...]) + b2_ref[...]
    h2 = _bn_swish(h2, g2_ref[...], be2_ref[...])
    h_ref[...] = h2
    xxr_ref[...] = jnp.sum(h2 * h2, axis=1).reshape(1, BS)


def _knn_body(h_blk_ref, h_all_ref, xxr_ref, idx_ref):
    hb = h_blk_ref[...]
    gram = _dot_t(hb, h_all_ref[...])
    inner = -2.0 * gram
    xxb = jnp.sum(hb * hb, axis=1, keepdims=True)
    s = (-xxb - inner) - xxr_ref[...]
    iota = lax.broadcasted_iota(jnp.int32, s.shape, 1)
    lane = lax.broadcasted_iota(jnp.int32, (s.shape[0], 128), 1)
    picks = []
    for k in range(K):
        m = jnp.max(s, axis=1, keepdims=True)
        cand = jnp.where(s == m, iota, BS)
        ik = jnp.min(cand, axis=1, keepdims=True)
        picks.append(ik)
        if k < K - 1:
            s = jnp.where(iota == ik, -jnp.inf, s)
    out = jnp.where(lane == 0, picks[0],
                    jnp.where(lane == 1, picks[1],
                              jnp.where(lane == 2, picks[2], 0)))
    idx_ref[...] = out


_NW = 32
_CHUNK = 48
_ROWS_PER_W = P3 // _NW
_NCHUNK = _ROWS_PER_W // _CHUNK
_HROWS_PER_W = BS // _NW
_NV = D // 16


def _smote_body(h_hbm, idxt_hbm, w_hbm, out_hbm,
                bidx_all, bidx_flat, w_all, b0, b1, o0, o1, cp_v, sem, semo):
    wid = lax.axis_index("s") * 2 + lax.axis_index("c")
    hbase = wid * _HROWS_PER_W
    sbase = wid * _ROWS_PER_W

    for t in range(3):
        pltpu.sync_copy(idxt_hbm.at[pl.ds(t * BS + hbase, _HROWS_PER_W)],
                        bidx_all.at[pl.ds(t * _HROWS_PER_W, _HROWS_PER_W)])
    pltpu.sync_copy(w_hbm.at[pl.ds(sbase * 16, _ROWS_PER_W * 16)], w_all)

    for c in range(_NCHUNK):
        for t in range(3):
            bidx_flat[pl.ds(c * _CHUNK + 16 * t, 16)] = \
                bidx_all[pl.ds(t * _HROWS_PER_W + c * 16, 16)]

    def issue(c, buf):
        return pltpu.async_copy(h_hbm.at[bidx_flat.at[pl.ds(c * _CHUNK,
                                                            _CHUNK)]],
                                buf, sem)

    def compute(c, buf, obuf):
        def anchor(aj, _):
            aregs = [buf[aj, pl.ds(16 * v, 16)] for v in range(_NV)]
            for v in range(_NV):
                obuf[aj * 3, pl.ds(16 * v, 16)] = aregs[v]
            for t in range(1, 3):
                j = aj * 3 + t
                wrow = w_all[pl.ds((c * _CHUNK + j) * 16, 16)]
                for v in range(_NV):
                    sl = pl.ds(16 * v, 16)
                    b = buf[16 * t + aj, sl]
                    obuf[j, sl] = aregs[v] + wrow * (b - aregs[v])
            return 0

        lax.fori_loop(0, 16, anchor, 0)
        return pltpu.async_copy(obuf,
                                out_hbm.at[pl.ds(BS + sbase + c * _CHUNK,
                                                 _CHUNK)], semo)

    def drain_gather(buf):
        pltpu.make_async_copy(h_hbm.at[pl.ds(0, _CHUNK)], buf, sem).wait()

    def drain_out(obuf):
        pltpu.make_async_copy(obuf, out_hbm.at[pl.ds(BS, _CHUNK)],
                              semo).wait()

    issue(0, b0)

    def pair(p, _):
        c0 = 2 * p
        issue(c0 + 1, b1)
        drain_gather(b0)

        @pl.when(p > 0)
        def _():
            drain_out(o0)

        compute(c0, b0, o0)

        @pl.when(p < _NCHUNK // 2 - 1)
        def _():
            issue(c0 + 2, b0)

        drain_gather(b1)

        @pl.when(p > 0)
        def _():
            drain_out(o1)

        compute(c0 + 1, b1, o1)
        return 0

    lax.fori_loop(0, _NCHUNK // 2, pair, 0)
    for q in range(4):
        pltpu.sync_copy(h_hbm.at[pl.ds(hbase + 32 * q, 32)], cp_v)
        pltpu.sync_copy(cp_v, out_hbm.at[pl.ds(hbase + 32 * q, 32)])
    drain_out(o0)
    drain_out(o1)


@jax.jit
def _pipeline(fea, lbl, W1, b1, g1, be1, W2, b2, g2, be2):
    vecs = [v.reshape(1, D) for v in (b1, g1, be1, b2, g2, be2)]
    b1r, g1r, be1r, b2r, g2r, be2r = vecs

    h, xxr = pl.pallas_call(
        _transform_body,
        out_shape=(jax.ShapeDtypeStruct((BS, D), jnp.float32),
                   jax.ShapeDtypeStruct((1, BS), jnp.float32)),
    )(fea, W1, b1r, g1r, be1r, W2, b2r, g2r, be2r)

    nblk = 16
    blk = BS // nblk
    idx_pad = pl.pallas_call(
        _knn_body,
        grid=(nblk,),
        in_specs=[
            pl.BlockSpec((blk, D), lambda i: (i, 0)),
            pl.BlockSpec((BS, D), lambda i: (0, 0)),
            pl.BlockSpec((1, BS), lambda i: (0, 0)),
        ],
        out_specs=pl.BlockSpec((blk, 128), lambda i: (i, 0)),
        out_shape=jax.ShapeDtypeStruct((BS, 128), jnp.int32),
    )(h, h, xxr)

    idxt = jnp.transpose(idx_pad[:, :K])

    mesh = plsc.VectorSubcoreMesh(core_axis_name="c", subcore_axis_name="s")
    smote = pl.kernel(
        _smote_body,
        mesh=mesh,
        out_type=jax.ShapeDtypeStruct((BS + P3, D), jnp.float32),
        scratch_types=[
            pltpu.VMEM((3 * _HROWS_PER_W,), jnp.int32),
            pltpu.VMEM((_ROWS_PER_W,), jnp.int32),
            pltpu.VMEM((_ROWS_PER_W * 16,), jnp.float32),
            pltpu.VMEM((_CHUNK, D), jnp.float32),
            pltpu.VMEM((_CHUNK, D), jnp.float32),
            pltpu.VMEM((_CHUNK, D), jnp.float32),
            pltpu.VMEM((_CHUNK, D), jnp.float32),
            pltpu.VMEM((32, D), jnp.float32),
            pltpu.SemaphoreType.DMA,
            pltpu.SemaphoreType.DMA,
        ],
    )
    fea_out = smote(h, idxt.reshape(-1), jnp.asarray(_W_SPLAT_NP.reshape(-1)))

    lbl_out = jnp.concatenate([lbl, jnp.ones((P3, 1), jnp.float32)], axis=0)
    return fea_out, lbl_out


def kernel(fea, lbl, W1, b1, g1, be1, W2, b2, g2, be2):
    return _pipeline(fea, lbl, W1, b1, g1, be1, W2, b2, g2, be2)


# speedup vs baseline: 1.9199x; 1.0001x over previous
import jax
import jax.numpy as jnp
import numpy as np
from jax import lax
from jax.experimental import pallas as pl
from jax.experimental.pallas import tpu as pltpu
from jax.experimental.pallas import tpu_sc as plsc

EPS = 1e-5
BS = 4096
D = 512
K = 3
P3 = BS * K

_W_NP = np.random.default_rng(0).random(P3).astype(np.float32)
_W_SPLAT_NP = np.repeat(_W_NP[:, None], 16, axis=1)

def _dot_t(a, b):
    return lax.dot_general(a.astype(jnp.bfloat16), b.astype(jnp.bfloat16),
                           (((1,), (1,)), ((), ())),
                           preferred_element_type=jnp.float32)


def _bn_swish(x, g, b):
    m = jnp.mean(x, axis=0, keepdims=True)
    v = jnp.mean((x - m) ** 2, axis=0, keepdims=True)
    y = (x - m) / jnp.sqrt(v + EPS) * g + b
    return y * jax.nn.sigmoid(y)


def _transform_body(fea_ref, w1_ref, b1_ref, g1_ref, be1_ref,
                    w2_ref, b2_ref, g2_ref, be2_ref, h_ref, xxr_ref):
    h1 = _dot_t(fea_ref[...], w1_ref[...]) + b1_ref[...]
    h1 = _bn_swish(h1, g1_ref[...], be1_ref[...])
    h2 = _dot_t(h1, w2_ref[...]) + b2_ref[...]
    h2 = _bn_swish(h2, g2_ref[...], be2_ref[...])
    h_ref[...] = h2
    xxr_ref[...] = jnp.sum(h2 * h2, axis=1).reshape(1, BS)


def _knn_body(h_blk_ref, h_all_ref, xxr_ref, idx_ref):
    hb = h_blk_ref[...]
    gram = _dot_t(hb, h_all_ref[...])
    inner = -2.0 * gram
    xxb = jnp.sum(hb * hb, axis=1, keepdims=True)
    s = (-xxb - inner) - xxr_ref[...]
    iota = lax.broadcasted_iota(jnp.int32, s.shape, 1)
    lane = lax.broadcasted_iota(jnp.int32, (s.shape[0], 128), 1)
    picks = []
    for k in range(K):
        m = jnp.max(s, axis=1, keepdims=True)
        cand = jnp.where(s == m, iota, BS)
        ik = jnp.min(cand, axis=1, keepdims=True)
        picks.append(ik)
        if k < K - 1:
            s = jnp.where(iota == ik, -jnp.inf, s)
    out = jnp.where(lane == 0, picks[0],
                    jnp.where(lane == 1, picks[1],
                              jnp.where(lane == 2, picks[2], 0)))
    idx_ref[...] = out


_NW = 32
_CHUNK = 48
_ROWS_PER_W = P3 // _NW
_NCHUNK = _ROWS_PER_W // _CHUNK
_HROWS_PER_W = BS // _NW
_NV = D // 16


def _smote_body(h_hbm, idxt_hbm, w_hbm, out_hbm,
                bidx_all, bidx_flat, w_all, b0, b1, o0, o1, cp_v, sem, semo):
    wid = lax.axis_index("s") * 2 + lax.axis_index("c")
    hbase = wid * _HROWS_PER_W
    sbase = wid * _ROWS_PER_W

    for t in range(3):
        pltpu.sync_copy(idxt_hbm.at[pl.ds(t * BS + hbase, _HROWS_PER_W)],
                        bidx_all.at[pl.ds(t * _HROWS_PER_W, _HROWS_PER_W)])
    pltpu.sync_copy(w_hbm.at[pl.ds(sbase * 16, _ROWS_PER_W * 16)], w_all)

    for c in range(_NCHUNK):
        for t in range(3):
            bidx_flat[pl.ds(c * _CHUNK + 16 * t, 16)] = \
                bidx_all[pl.ds(t * _HROWS_PER_W + c * 16, 16)]

    def issue(c, buf):
        return pltpu.async_copy(h_hbm.at[bidx_flat.at[pl.ds(c * _CHUNK,
                                                            _CHUNK)]],
                                buf, sem)

    def compute(c, buf, obuf):
        def anchor(aj, _):
            aregs = [buf[aj, pl.ds(16 * v, 16)] for v in range(_NV)]
            for v in range(_NV):
                obuf[aj * 3, pl.ds(16 * v, 16)] = aregs[v]
            for t in range(1, 3):
                j = aj * 3 + t
                wrow = w_all[pl.ds((c * _CHUNK + j) * 16, 16)]
                for v in range(_NV):
                    sl = pl.ds(16 * v, 16)
                    b = buf[16 * t + aj, sl]
                    obuf[j, sl] = aregs[v] + wrow * (b - aregs[v])
            return 0

        lax.fori_loop(0, 16, anchor, 0)
        return pltpu.async_copy(obuf,
                                out_hbm.at[pl.ds(BS + sbase + c * _CHUNK,
                                                 _CHUNK)], semo)

    def drain_gather(buf):
        pltpu.make_async_copy(h_hbm.at[pl.ds(0, _CHUNK)], buf, sem).wait()

    def drain_out(obuf):
        pltpu.make_async_copy(obuf, out_hbm.at[pl.ds(BS, _CHUNK)],
                              semo).wait()

    issue(0, b0)

    def pair(p, _):
        c0 = 2 * p
        issue(c0 + 1, b1)
        drain_gather(b0)

        @pl.when(p > 0)
        def _():
            drain_out(o0)

        compute(c0, b0, o0)

        @pl.when(p < _NCHUNK // 2 - 1)
        def _():
            issue(c0 + 2, b0)

        drain_gather(b1)

        @pl.when(p > 0)
        def _():
            drain_out(o1)

        compute(c0 + 1, b1, o1)
        return 0

    lax.fori_loop(0, _NCHUNK // 2, pair, 0)
    for q in range(4):
        pltpu.sync_copy(h_hbm.at[pl.ds(hbase + 32 * q, 32)], cp_v)
        pltpu.sync_copy(cp_v, out_hbm.at[pl.ds(hbase + 32 * q, 32)])
    drain_out(o0)
    drain_out(o1)


@jax.jit
def _pipeline(fea, lbl, W1, b1, g1, be1, W2, b2, g2, be2):
    vecs = [v.reshape(1, D) for v in (b1, g1, be1, b2, g2, be2)]
    b1r, g1r, be1r, b2r, g2r, be2r = vecs

    h, xxr = pl.pallas_call(
        _transform_body,
        out_shape=(jax.ShapeDtypeStruct((BS, D), jnp.float32),
                   jax.ShapeDtypeStruct((1, BS), jnp.float32)),
    )(fea, W1, b1r, g1r, be1r, W2, b2r, g2r, be2r)

    nblk = 16
    blk = BS // nblk
    idx_pad = pl.pallas_call(
        _knn_body,
        grid=(nblk,),
        in_specs=[
            pl.BlockSpec((blk, D), lambda i: (i, 0)),
            pl.BlockSpec((BS, D), lambda i: (0, 0)),
            pl.BlockSpec((1, BS), lambda i: (0, 0)),
        ],
        out_specs=pl.BlockSpec((blk, 128), lambda i: (i, 0)),
        out_shape=jax.ShapeDtypeStruct((BS, 128), jnp.int32),
    )(h, h, xxr)

    idxt = jnp.transpose(idx_pad[:, :K])

    mesh = plsc.VectorSubcoreMesh(core_axis_name="c", subcore_axis_name="s")
    smote = pl.kernel(
        _smote_body,
        mesh=mesh,
        out_type=jax.ShapeDtypeStruct((BS + P3, D), jnp.float32),
        scratch_types=[
            pltpu.VMEM((3 * _HROWS_PER_W,), jnp.int32),
            pltpu.VMEM((_ROWS_PER_W,), jnp.int32),
            pltpu.VMEM((_ROWS_PER_W * 16,), jnp.float32),
            pltpu.VMEM((_CHUNK, D), jnp.float32),
            pltpu.VMEM((_CHUNK, D), jnp.float32),
            pltpu.VMEM((_CHUNK, D), jnp.float32),
            pltpu.VMEM((_CHUNK, D), jnp.float32),
            pltpu.VMEM((32, D), jnp.float32),
            pltpu.SemaphoreType.DMA,
            pltpu.SemaphoreType.DMA,
        ],
    )
    fea_out = smote(h, idxt.reshape(-1), jnp.asarray(_W_SPLAT_NP.reshape(-1)))

    lbl_out = jnp.concatenate([lbl, jnp.ones((P3, 1), jnp.float32)], axis=0)
    return fea_out, lbl_out


def kernel(fea, lbl, W1, b1, g1, be1, W2, b2, g2, be2):
    return _pipeline(fea, lbl, W1, b1, g1, be1, W2, b2, g2, be2)
